# Initial kernel scaffold; baseline (speedup 1.0000x reference)
import jax
import jax.numpy as jnp
from jax.experimental import pallas as pl


def kernel(x, W, b):
    raise NotImplementedError("write your pallas kernel here")


# speedup vs baseline: 9.2439x; 9.2439x over previous
import functools

import jax
import jax.numpy as jnp
from jax.experimental import pallas as pl
from jax.experimental.pallas import tpu as pltpu

_K = 8
_BIG = 2**30


def _router_body(x_ref, w_ref, b_ref, idx_ref, sc_ref, vals_scr, gidx_scr):
    j = pl.program_id(1)
    nj = pl.num_programs(1)
    t_tile, _ = x_ref.shape
    p_tile = w_ref.shape[0]

    s = jax.lax.dot_general(
        x_ref[...], w_ref[...], (((1,), (1,)), ((), ())),
        preferred_element_type=jnp.float32,
        precision=jax.lax.Precision.DEFAULT,
    )
    s = s + b_ref[pl.ds(j * p_tile, p_tile)][None, :]
    sc_ref[...] = s

    @pl.when(j == 0)
    def _init():
        vals_scr[...] = jnp.full_like(vals_scr, -jnp.inf)
        gidx_scr[...] = jnp.zeros_like(gidx_scr)

    cvals = jnp.concatenate([vals_scr[...], s], axis=1)
    blk_gidx = j * p_tile + jax.lax.broadcasted_iota(
        jnp.int32, (t_tile, p_tile), 1)
    cidx = jnp.concatenate([gidx_scr[...], blk_gidx], axis=1)
    pos_iota = jax.lax.broadcasted_iota(jnp.int32, cvals.shape, 1)

    new_vals = []
    new_idx = []
    for _ in range(_K):
        m = jnp.max(cvals, axis=1, keepdims=True)
        pos = jnp.min(jnp.where(cvals == m, pos_iota, _BIG), axis=1,
                      keepdims=True)
        sel = pos_iota == pos
        gi = jnp.max(jnp.where(sel, cidx, -1), axis=1, keepdims=True)
        new_vals.append(m)
        new_idx.append(gi)
        cvals = jnp.where(sel, -jnp.inf, cvals)
    vals_scr[...] = jnp.concatenate(new_vals, axis=1)
    gidx_scr[...] = jnp.concatenate(new_idx, axis=1)

    @pl.when(j == nj - 1)
    def _fin():
        idx_ref[...] = gidx_scr[...]


@functools.partial(jax.jit, static_argnames=("interpret",))
def _router(x2d, w, b, interpret=False):
    t, d = x2d.shape
    p = w.shape[0]
    t_tile = min(512, t)
    p_tile = min(512, p)
    grid = (t // t_tile, p // p_tile)
    idx_out, scores = pl.pallas_call(
        _router_body,
        grid=grid,
        in_specs=[
            pl.BlockSpec((t_tile, d), lambda i, j: (i, 0)),
            pl.BlockSpec((p_tile, d), lambda i, j: (j, 0)),
            pl.BlockSpec((p,), lambda i, j: (0,)),
        ],
        out_specs=[
            pl.BlockSpec((t_tile, _K), lambda i, j: (i, 0)),
            pl.BlockSpec((t_tile, p_tile), lambda i, j: (i, j)),
        ],
        out_shape=[
            jax.ShapeDtypeStruct((t, _K), jnp.int32),
            jax.ShapeDtypeStruct((t, p), jnp.float32),
        ],
        scratch_shapes=[
            pltpu.VMEM((t_tile, _K), jnp.float32),
            pltpu.VMEM((t_tile, _K), jnp.int32),
        ],
        compiler_params=pltpu.CompilerParams(
            dimension_semantics=("parallel", "arbitrary"),
        ),
        interpret=interpret,
    )(x2d, w, b)
    return idx_out, scores


def kernel(x, w, b):
    bsz, seq, d = x.shape
    p = w.shape[0]
    x2d = x.reshape(bsz * seq, d)
    idx_out, scores = _router(x2d, w, b)
    return idx_out.reshape(bsz, seq, _K), scores.reshape(bsz, seq, p)


# speedup vs baseline: 11.9898x; 1.2970x over previous
import functools

import jax
import jax.numpy as jnp
from jax.experimental import pallas as pl
from jax.experimental.pallas import tpu as pltpu

_K = 8
_BIG = 2**30


def _merge_top8(sb, base, t_tile, p_tile, vals_scr, gidx_scr):
    cvals = jnp.concatenate([vals_scr[...], sb], axis=1)
    blk_gidx = base + jax.lax.broadcasted_iota(
        jnp.int32, (t_tile, p_tile), 1)
    cidx = jnp.concatenate([gidx_scr[...], blk_gidx], axis=1)
    new_vals = []
    new_idx = []
    for _ in range(_K):
        m = jnp.max(cvals, axis=1, keepdims=True)
        gi = jnp.min(jnp.where(cvals == m, cidx, _BIG), axis=1, keepdims=True)
        cvals = jnp.where(cidx == gi, -jnp.inf, cvals)
        new_vals.append(m)
        new_idx.append(gi)
    vals_scr[...] = jnp.concatenate(new_vals, axis=1)
    gidx_scr[...] = jnp.concatenate(new_idx, axis=1)


def _router_body(x_ref, w_ref, b_ref, idx_ref, sc_ref, vals_scr, gidx_scr,
                 sprev_scr):
    j = pl.program_id(1)
    nj = pl.num_programs(1)
    t_tile, _ = x_ref.shape
    p_tile = w_ref.shape[0]

    s = jax.lax.dot_general(
        x_ref[...], w_ref[...], (((1,), (1,)), ((), ())),
        preferred_element_type=jnp.float32,
        precision=jax.lax.Precision.DEFAULT,
    )
    s = s + b_ref[pl.ds(j * p_tile, p_tile)][None, :]
    sc_ref[...] = s

    @pl.when(j == 0)
    def _init():
        vals_scr[...] = jnp.full_like(vals_scr, -jnp.inf)
        gidx_scr[...] = jnp.zeros_like(gidx_scr)

    @pl.when(j > 0)
    def _merge_prev():
        _merge_top8(sprev_scr[...], (j - 1) * p_tile, t_tile, p_tile,
                    vals_scr, gidx_scr)

    sprev_scr[...] = s

    @pl.when(j == nj - 1)
    def _fin():
        _merge_top8(s, j * p_tile, t_tile, p_tile, vals_scr, gidx_scr)
        idx_ref[...] = gidx_scr[...]


@functools.partial(jax.jit, static_argnames=("interpret",))
def _router(x2d, w, b, interpret=False):
    t, d = x2d.shape
    p = w.shape[0]
    t_tile = min(512, t)
    p_tile = min(512, p)
    grid = (t // t_tile, p // p_tile)
    idx_out, scores = pl.pallas_call(
        _router_body,
        grid=grid,
        in_specs=[
            pl.BlockSpec((t_tile, d), lambda i, j: (i, 0)),
            pl.BlockSpec((p_tile, d), lambda i, j: (j, 0)),
            pl.BlockSpec((p,), lambda i, j: (0,)),
        ],
        out_specs=[
            pl.BlockSpec((t_tile, _K), lambda i, j: (i, 0)),
            pl.BlockSpec((t_tile, p_tile), lambda i, j: (i, j)),
        ],
        out_shape=[
            jax.ShapeDtypeStruct((t, _K), jnp.int32),
            jax.ShapeDtypeStruct((t, p), jnp.float32),
        ],
        scratch_shapes=[
            pltpu.VMEM((t_tile, _K), jnp.float32),
            pltpu.VMEM((t_tile, _K), jnp.int32),
            pltpu.VMEM((t_tile, p_tile), jnp.float32),
        ],
        compiler_params=pltpu.CompilerParams(
            dimension_semantics=("parallel", "arbitrary"),
        ),
        interpret=interpret,
    )(x2d, w, b)
    return idx_out, scores


def kernel(x, w, b):
    bsz, seq, d = x.shape
    p = w.shape[0]
    x2d = x.reshape(bsz * seq, d)
    idx_out, scores = _router(x2d, w, b)
    return idx_out.reshape(bsz, seq, _K), scores.reshape(bsz, seq, p)
